# baseline (speedup 1.0000x reference)
import functools

import jax
import jax.numpy as jnp
from jax.experimental import pallas as pl
import jax.experimental.pallas.tpu as pltpu


_N_CHUNKS = 8


def _blk_kernel(idx_ref, w_ref, x_ref, b_ref, out_ref, xbf_ref, stage0, stage1,
                sems, *, in_features, bo):
    fan = idx_ref.shape[0]
    batch = xbf_ref.shape[0]
    rows = batch // _N_CHUNKS
    stages = [stage0, stage1]

    @pl.when(pl.program_id(0) == 0)
    def _cast_x():
        copies = [None, None]
        for c in range(_N_CHUNKS):
            b = c % 2
            copies[b] = pltpu.make_async_copy(
                x_ref.at[pl.ds(c * rows, rows), :], stages[b], sems.at[b]
            )
            copies[b].start()
            if c > 0:
                prev = 1 - b
                copies[prev].wait()
                xbf_ref[pl.ds((c - 1) * rows, rows), :] = (
                    stages[prev][...].astype(jnp.bfloat16)
                )
        copies[(_N_CHUNKS - 1) % 2].wait()
        xbf_ref[pl.ds((_N_CHUNKS - 1) * rows, rows), :] = (
            stages[(_N_CHUNKS - 1) % 2][...].astype(jnp.bfloat16)
        )

    iota = jax.lax.broadcasted_iota(jnp.int16, (in_features, bo), 0)
    idx = idx_ref[...]
    w = w_ref[...]
    s = jnp.zeros((in_features, bo), jnp.bfloat16)
    for f in range(fan):
        s = jnp.where(iota == idx[f : f + 1, :], w[f : f + 1, :], s)
    out_ref[...] = (
        jnp.dot(xbf_ref[...], s, preferred_element_type=jnp.float32)
        + b_ref[...]
    )


def kernel(input, weight, bias, indx_seqs):
    batch, in_features = input.shape
    out_features, fan_in = weight.shape
    bo = min(256, out_features)
    n_blk = out_features // bo

    idx_t = indx_seqs.astype(jnp.int16).T
    w_t = weight.T.astype(jnp.bfloat16)
    bias2 = bias.reshape(1, out_features)

    out = pl.pallas_call(
        functools.partial(_blk_kernel, in_features=in_features, bo=bo),
        grid=(n_blk,),
        in_specs=[
            pl.BlockSpec((fan_in, bo), lambda j: (0, j)),
            pl.BlockSpec((fan_in, bo), lambda j: (0, j)),
            pl.BlockSpec(memory_space=pltpu.MemorySpace.HBM),
            pl.BlockSpec((1, bo), lambda j: (0, j)),
        ],
        out_specs=pl.BlockSpec((batch, bo), lambda j: (0, j)),
        out_shape=jax.ShapeDtypeStruct((batch, out_features), jnp.float32),
        scratch_shapes=[
            pltpu.VMEM((batch, in_features), jnp.bfloat16),
            pltpu.VMEM((batch // _N_CHUNKS, in_features), jnp.float32),
            pltpu.VMEM((batch // _N_CHUNKS, in_features), jnp.float32),
            pltpu.SemaphoreType.DMA((2,)),
        ],
    )(idx_t, w_t, input, bias2)
    return out


# speedup vs baseline: 1.0118x; 1.0118x over previous
import functools

import jax
import jax.numpy as jnp
from jax.experimental import pallas as pl
import jax.experimental.pallas.tpu as pltpu


def _blk_kernel(idx_ref, w_ref, x_ref, b_ref, out_ref, s_ref, *,
                in_features, out_features):
    fan = idx_ref.shape[0]
    bo = 256

    @pl.when(pl.program_id(0) == 0)
    def _build_s():
        iota = jax.lax.broadcasted_iota(jnp.int16, (in_features, bo), 0)
        for blk in range(out_features // bo):
            idx = idx_ref[:, blk * bo : (blk + 1) * bo]
            w = w_ref[:, blk * bo : (blk + 1) * bo]
            s = jnp.zeros((in_features, bo), jnp.bfloat16)
            for f in range(fan):
                s = jnp.where(iota == idx[f : f + 1, :], w[f : f + 1, :], s)
            s_ref[:, blk * bo : (blk + 1) * bo] = s

    out_ref[...] = (
        jnp.dot(
            x_ref[...].astype(jnp.bfloat16),
            s_ref[...],
            preferred_element_type=jnp.float32,
        )
        + b_ref[...]
    )


def kernel(input, weight, bias, indx_seqs):
    batch, in_features = input.shape
    out_features, fan_in = weight.shape
    bb = min(512, batch)
    n_blk = batch // bb

    idx_t = indx_seqs.astype(jnp.int16).T
    w_t = weight.T.astype(jnp.bfloat16)
    bias2 = bias.reshape(1, out_features)

    out = pl.pallas_call(
        functools.partial(
            _blk_kernel, in_features=in_features, out_features=out_features
        ),
        grid=(n_blk,),
        in_specs=[
            pl.BlockSpec((fan_in, out_features), lambda j: (0, 0)),
            pl.BlockSpec((fan_in, out_features), lambda j: (0, 0)),
            pl.BlockSpec((bb, in_features), lambda j: (j, 0)),
            pl.BlockSpec((1, out_features), lambda j: (0, 0)),
        ],
        out_specs=pl.BlockSpec((bb, out_features), lambda j: (j, 0)),
        out_shape=jax.ShapeDtypeStruct((batch, out_features), jnp.float32),
        scratch_shapes=[
            pltpu.VMEM((in_features, out_features), jnp.bfloat16),
        ],
    )(idx_t, w_t, input, bias2)
    return out
